# baseline (speedup 1.0000x reference)
import functools

import jax
import jax.numpy as jnp
import numpy as np
from jax import lax
from jax.experimental import pallas as pl
from jax.experimental.pallas import tpu as pltpu
from jax.experimental.pallas import tpu_sc as plsc

_N = 10000
_E = 320000
_H = 128
_OUT = 2
_MNN = 1000
_L = 4
_EPS = 1e-5

_C = 128
_EPAD = 327680
_NCHUNK = _EPAD // _C
_NS = 16
_NC = 2
_CPT = _NCHUNK // (_NC * _NS)
_RPT = 632
_NROW = _NS * _RPT
_DW = 16


_MESH = plsc.VectorSubcoreMesh(core_axis_name="c", subcore_axis_name="s")

_PAD_N = _EPAD - _E
_PAD_PACKED = (np.arange(_PAD_N, dtype=np.int64) % _N
               | ((np.arange(_PAD_N, dtype=np.int64) % (_NROW - _N) + _N)
                  << 16)).astype(np.int32)


def _sc_agg_body(x_hbm, packed_hbm, agg_hbm,
                 packed_v, srcr, dstr, rows0, rows1, acc_sh, sem0, sem1):
    cid = lax.axis_index("c")
    sid = lax.axis_index("s")
    rows = pl.ds(sid * _RPT, _RPT)
    chunks = pl.ds((cid * _NS + sid) * _CPT, _CPT)

    pltpu.sync_copy(packed_hbm.at[chunks], packed_v)

    z16 = jnp.zeros((16,), jnp.float32)

    def zrow(r, carry):
        for c in range(_H // 16):
            rows0[r, pl.ds(c * 16, 16)] = z16
        return carry

    lax.fori_loop(0, _C, zrow, 0, unroll=False)
    for j in range(5):
        n = _C if j < 4 else _RPT - 4 * _C
        pltpu.sync_copy(rows0.at[pl.ds(0, n)],
                        acc_sh.at[pl.ds(sid * _RPT + j * _C, n)])
    plsc.subcore_barrier()

    def unpack(k, slot):
        for c in range(_C // 16):
            w = packed_v[k, pl.ds(c * 16, 16)]
            srcr[slot, pl.ds(c * 16, 16)] = w & 0xFFFF
            dstr[slot, pl.ds(c * 16, 16)] = lax.shift_right_logical(w, 16)

    def gather(slot, buf, sem):
        pltpu.async_copy(x_hbm.at[srcr.at[slot]], buf, sem)

    def gwait(buf, sem):
        pltpu.make_async_copy(x_hbm.at[srcr.at[0]], buf, sem).wait()

    def scatter(buf, slot):
        pltpu.sync_copy(buf, acc_sh.at[dstr.at[slot]], add=True)

    unpack(0, 0)
    gather(0, rows0, sem0)

    def body2(i, carry):
        k0 = 2 * i
        unpack(k0 + 1, 1)
        gather(1, rows1, sem1)
        gwait(rows0, sem0)
        scatter(rows0, 0)

        @pl.when(i < _CPT // 2 - 1)
        def _():
            unpack(k0 + 2, 0)
            gather(0, rows0, sem0)

        gwait(rows1, sem1)
        scatter(rows1, 1)
        return carry

    lax.fori_loop(0, _CPT // 2, body2, 0, unroll=False)
    plsc.subcore_barrier()

    pltpu.sync_copy(acc_sh.at[rows], agg_hbm.at[cid, rows])


_sc_agg = pl.kernel(
    _sc_agg_body,
    out_type=jax.ShapeDtypeStruct((_NC, _NROW, _H), jnp.float32),
    mesh=_MESH,
    scratch_types=(
        pltpu.VMEM((_CPT, _C), jnp.int32),
        pltpu.VMEM((2, _C), jnp.int32),
        pltpu.VMEM((2, _C), jnp.int32),
        pltpu.VMEM((_C, _H), jnp.float32),
        pltpu.VMEM((_C, _H), jnp.float32),
        pltpu.VMEM_SHARED((_NROW, _H), jnp.float32),
        pltpu.SemaphoreType.DMA,
        pltpu.SemaphoreType.DMA,
    ),
)


def _sc_deg_body(packed_hbm, zerosd_hbm, ones_hbm, deg_hbm,
                 packed_v, dstr, ones_v, deg_sh, gsem):
    cid = lax.axis_index("c")
    sid = lax.axis_index("s")
    rows = pl.ds(sid * _RPT, _RPT)
    chunks = pl.ds((cid * _NS + sid) * _CPT, _CPT)

    pltpu.sync_copy(packed_hbm.at[chunks], packed_v)
    pltpu.sync_copy(zerosd_hbm, deg_sh.at[rows])
    pltpu.sync_copy(ones_hbm, ones_v)
    plsc.subcore_barrier()

    def chunk(k, carry):
        for c in range(_C // 16):
            w = packed_v[k, pl.ds(c * 16, 16)]
            dstr[0, pl.ds(c * 16, 16)] = lax.shift_right_logical(w, 16)
        pltpu.sync_copy(ones_v, deg_sh.at[dstr.at[0]], add=True)
        return carry

    lax.fori_loop(0, _CPT, chunk, 0, unroll=False)
    plsc.subcore_barrier()
    pltpu.sync_copy(deg_sh.at[rows], deg_hbm.at[cid, rows])


_sc_deg = pl.kernel(
    _sc_deg_body,
    out_type=jax.ShapeDtypeStruct((_NC, _NROW, _DW), jnp.float32),
    mesh=_MESH,
    compiler_params=pltpu.CompilerParams(use_tc_tiling_on_sc=False),
    scratch_types=(
        pltpu.VMEM((_CPT, _C), jnp.int32),
        pltpu.VMEM((1, _C), jnp.int32),
        pltpu.VMEM((_C, _DW), jnp.float32),
        pltpu.VMEM_SHARED((_NROW, _DW), jnp.float32),
        pltpu.SemaphoreType.DMA,
    ),
)


def _tc_concat4_body(a, b, c, d, out):
    out[:, 0:32] = a[...]
    out[:, 32:64] = b[...]
    out[:, 64:96] = c[...]
    out[:, 96:128] = d[...]


_tc_concat4 = pl.pallas_call(
    _tc_concat4_body,
    out_shape=jax.ShapeDtypeStruct((_N, _H), jnp.float32),
)


def _tc_pre_body(x_ref, wr, b, out):
    out[...] = (jnp.dot(x_ref[...], wr[...],
                        preferred_element_type=jnp.float32) + b[...])


_tc_pre = pl.pallas_call(
    _tc_pre_body,
    out_shape=jax.ShapeDtypeStruct((_N, _H), jnp.float32),
)


def _tc_post_body(aggp, yr, degn, wl, g, bb, out):
    agg = aggp[0, :_N, :] + aggp[1, :_N, :]
    deg = jnp.maximum(degn[0] + degn[1], 1.0)
    y = (jnp.dot(agg / deg, wl[...], preferred_element_type=jnp.float32)
         + yr[...])
    mu = jnp.mean(y, axis=0, keepdims=True)
    var = jnp.mean((y - mu) ** 2, axis=0, keepdims=True)
    y = (y - mu) * lax.rsqrt(var + _EPS) * g[...] + bb[...]
    out[...] = jnp.maximum(y, 0.0)


_tc_post = pl.pallas_call(
    _tc_post_body,
    out_shape=jax.ShapeDtypeStruct((_N, _H), jnp.float32),
)


def _tc_head1_body(x2d, w1, b1, w2, b2, out):
    h = jnp.dot(x2d[...], w1[...], preferred_element_type=jnp.float32)
    h = jnp.maximum(h + b1[...], 0.0)
    out[...] = jnp.dot(h, w2[...], preferred_element_type=jnp.float32) + b2[...]


_tc_head1 = pl.pallas_call(
    _tc_head1_body,
    out_shape=jax.ShapeDtypeStruct((_N * _H // _MNN, 1), jnp.float32),
)


def _tc_head2_body(t, g, bb, w1, b1, w2, b2, out):
    x = t[...]
    mu = jnp.mean(x, axis=0, keepdims=True)
    var = jnp.mean((x - mu) ** 2, axis=0, keepdims=True)
    x = (x - mu) * lax.rsqrt(var + _EPS) * g[...] + bb[...]
    x = jnp.maximum(x, 0.0)
    h = jnp.maximum(
        jnp.dot(x, w1[...], preferred_element_type=jnp.float32) + b1[...], 0.0)
    out[...] = (jnp.dot(h, w2[...], preferred_element_type=jnp.float32)
                + b2[...])


_tc_head2 = pl.pallas_call(
    _tc_head2_body,
    out_shape=jax.ShapeDtypeStruct((_N * _H // _MNN // _H, _OUT), jnp.float32),
)


def kernel(x_ori, gamora0, gamora1, gamora2, edge_index, Wl, bl, Wr, bn_g,
           bn_b, m1_W1, m1_b1, m1_W2, m1_b2, bn2_g, bn2_b, m2_W1, m2_b1,
           m2_W2, m2_b2):
    x = _tc_concat4(x_ori, gamora0, gamora1, gamora2)

    packed = jnp.concatenate(
        [jnp.bitwise_or(edge_index[0], jnp.left_shift(edge_index[1], 16)),
         jnp.asarray(_PAD_PACKED)]).reshape(_NCHUNK, _C)
    zerosd = jnp.zeros((_RPT, _DW), jnp.float32)
    ones = jnp.ones((_C, _DW), jnp.float32)

    degw = _sc_deg(packed, zerosd, ones)
    degn = degw[:, :_N, 0:1]
    for i in range(_L):
        aggp = _sc_agg(x, packed)
        yr = _tc_pre(x, Wr[i], bl[i].reshape(1, _H))
        x = _tc_post(aggp, yr, degn, Wl[i], bn_g[i].reshape(1, _H),
                     bn_b[i].reshape(1, _H))

    x2d = x.reshape(_N * _H // _MNN, _MNN)
    t = _tc_head1(x2d, m1_W1, m1_b1.reshape(1, _H), m1_W2,
                  m1_b2.reshape(1, 1))
    t10 = t.reshape(_N * _H // _MNN // _H, _H)
    out = _tc_head2(t10, bn2_g.reshape(1, _H), bn2_b.reshape(1, _H),
                    m2_W1, m2_b1.reshape(1, _H), m2_W2,
                    m2_b2.reshape(1, _OUT))
    return out


# speedup vs baseline: 1.0308x; 1.0308x over previous
import functools

import jax
import jax.numpy as jnp
import numpy as np
from jax import lax
from jax.experimental import pallas as pl
from jax.experimental.pallas import tpu as pltpu
from jax.experimental.pallas import tpu_sc as plsc

_N = 10000
_E = 320000
_H = 128
_OUT = 2
_MNN = 1000
_L = 4
_EPS = 1e-5

_C = 128
_EPAD = 327680
_NCHUNK = _EPAD // _C
_NS = 16
_NC = 2
_CPT = _NCHUNK // (_NC * _NS)
_RPT = 632
_NROW = _NS * _RPT
_DW = 16


_MESH = plsc.VectorSubcoreMesh(core_axis_name="c", subcore_axis_name="s")

_PAD_N = _EPAD - _E
_PAD_PACKED = (np.arange(_PAD_N, dtype=np.int64) % _N
               | ((np.arange(_PAD_N, dtype=np.int64) % (_NROW - _N) + _N)
                  << 16)).astype(np.int32)


def _sc_agg_body(x_hbm, packed_hbm, agg_hbm,
                 packed_v, srcr, dstr, rows0, rows1, acc_sh, sem0, sem1):
    cid = lax.axis_index("c")
    sid = lax.axis_index("s")
    rows = pl.ds(sid * _RPT, _RPT)
    chunks = pl.ds((cid * _NS + sid) * _CPT * _C, _CPT * _C)

    pltpu.sync_copy(packed_hbm.at[chunks], packed_v)

    z16 = jnp.zeros((16,), jnp.float32)

    def zrow(r, carry):
        for c in range(_H // 16):
            rows0[r, pl.ds(c * 16, 16)] = z16
        return carry

    lax.fori_loop(0, _C, zrow, 0, unroll=False)
    for j in range(5):
        n = _C if j < 4 else _RPT - 4 * _C
        pltpu.sync_copy(rows0.at[pl.ds(0, n)],
                        acc_sh.at[pl.ds(sid * _RPT + j * _C, n)])
    plsc.subcore_barrier()

    def unpack(k, slot):
        for c in range(_C // 16):
            w = packed_v[pl.ds(k * _C + c * 16, 16)]
            srcr[slot, pl.ds(c * 16, 16)] = w & 0xFFFF
            dstr[slot, pl.ds(c * 16, 16)] = lax.shift_right_logical(w, 16)

    def gather(slot, buf, sem):
        pltpu.async_copy(x_hbm.at[srcr.at[slot]], buf, sem)

    def gwait(buf, sem):
        pltpu.make_async_copy(x_hbm.at[srcr.at[0]], buf, sem).wait()

    def scatter(buf, slot):
        pltpu.sync_copy(buf, acc_sh.at[dstr.at[slot]], add=True)

    unpack(0, 0)
    gather(0, rows0, sem0)

    def body2(i, carry):
        k0 = 2 * i
        unpack(k0 + 1, 1)
        gather(1, rows1, sem1)
        gwait(rows0, sem0)
        scatter(rows0, 0)

        @pl.when(i < _CPT // 2 - 1)
        def _():
            unpack(k0 + 2, 0)
            gather(0, rows0, sem0)

        gwait(rows1, sem1)
        scatter(rows1, 1)
        return carry

    lax.fori_loop(0, _CPT // 2, body2, 0, unroll=False)
    plsc.subcore_barrier()

    pltpu.sync_copy(acc_sh.at[rows], agg_hbm.at[cid, rows])


_sc_agg = pl.kernel(
    _sc_agg_body,
    out_type=jax.ShapeDtypeStruct((_NC, _NROW, _H), jnp.float32),
    mesh=_MESH,
    scratch_types=(
        pltpu.VMEM((_CPT * _C,), jnp.int32),
        pltpu.VMEM((2, _C), jnp.int32),
        pltpu.VMEM((2, _C), jnp.int32),
        pltpu.VMEM((_C, _H), jnp.float32),
        pltpu.VMEM((_C, _H), jnp.float32),
        pltpu.VMEM_SHARED((_NROW, _H), jnp.float32),
        pltpu.SemaphoreType.DMA,
        pltpu.SemaphoreType.DMA,
    ),
)


def _sc_deg_body(packed_hbm, zerosd_hbm, ones_hbm, deg_hbm,
                 packed_v, dstr, ones_v, deg_sh, gsem):
    cid = lax.axis_index("c")
    sid = lax.axis_index("s")
    rows = pl.ds(sid * _RPT, _RPT)
    chunks = pl.ds((cid * _NS + sid) * _CPT * _C, _CPT * _C)

    pltpu.sync_copy(packed_hbm.at[chunks], packed_v)
    pltpu.sync_copy(zerosd_hbm, deg_sh.at[rows])
    pltpu.sync_copy(ones_hbm, ones_v)
    plsc.subcore_barrier()

    def chunk(k, carry):
        for c in range(_C // 16):
            w = packed_v[pl.ds(k * _C + c * 16, 16)]
            dstr[0, pl.ds(c * 16, 16)] = lax.shift_right_logical(w, 16)
        pltpu.sync_copy(ones_v, deg_sh.at[dstr.at[0]], add=True)
        return carry

    lax.fori_loop(0, _CPT, chunk, 0, unroll=False)
    plsc.subcore_barrier()
    pltpu.sync_copy(deg_sh.at[rows], deg_hbm.at[cid, rows])


_sc_deg = pl.kernel(
    _sc_deg_body,
    out_type=jax.ShapeDtypeStruct((_NC, _NROW, _DW), jnp.float32),
    mesh=_MESH,
    compiler_params=pltpu.CompilerParams(use_tc_tiling_on_sc=False),
    scratch_types=(
        pltpu.VMEM((_CPT * _C,), jnp.int32),
        pltpu.VMEM((1, _C), jnp.int32),
        pltpu.VMEM((_C, _DW), jnp.float32),
        pltpu.VMEM_SHARED((_NROW, _DW), jnp.float32),
        pltpu.SemaphoreType.DMA,
    ),
)


def _tc_pack_body(ei, out):
    out[...] = jnp.bitwise_or(ei[0, :], jnp.left_shift(ei[1, :], 16))


_tc_pack = pl.pallas_call(
    _tc_pack_body,
    out_shape=jax.ShapeDtypeStruct((_E,), jnp.int32),
)


def _tc_concat4_body(a, b, c, d, out):
    out[:, 0:32] = a[...]
    out[:, 32:64] = b[...]
    out[:, 64:96] = c[...]
    out[:, 96:128] = d[...]


_tc_concat4 = pl.pallas_call(
    _tc_concat4_body,
    out_shape=jax.ShapeDtypeStruct((_N, _H), jnp.float32),
)


def _tc_pre_body(x_ref, wr, b, out):
    out[...] = (jnp.dot(x_ref[...], wr[...],
                        preferred_element_type=jnp.float32) + b[...])


_tc_pre = pl.pallas_call(
    _tc_pre_body,
    out_shape=jax.ShapeDtypeStruct((_N, _H), jnp.float32),
)


def _tc_post_body(aggp, yr, degn, wl, g, bb, out):
    agg = aggp[0, :_N, :] + aggp[1, :_N, :]
    deg = jnp.maximum(degn[0] + degn[1], 1.0)
    y = (jnp.dot(agg / deg, wl[...], preferred_element_type=jnp.float32)
         + yr[...])
    mu = jnp.mean(y, axis=0, keepdims=True)
    var = jnp.mean((y - mu) ** 2, axis=0, keepdims=True)
    y = (y - mu) * lax.rsqrt(var + _EPS) * g[...] + bb[...]
    out[...] = jnp.maximum(y, 0.0)


_tc_post = pl.pallas_call(
    _tc_post_body,
    out_shape=jax.ShapeDtypeStruct((_N, _H), jnp.float32),
)


_B = _N * _H // _MNN // _H


def _tc_head_body(x2d, w1, b1, w2, b2, g, bb, v1, c1, v2, c2, out):
    h = jnp.dot(x2d[...], w1[...], preferred_element_type=jnp.float32)
    h = jnp.maximum(h + b1[...], 0.0)
    h3 = h.reshape(_B, _H, _H)
    t = jnp.sum(h3 * w2[0, :], axis=-1) + b2[...]
    mu = jnp.mean(t, axis=0, keepdims=True)
    var = jnp.mean((t - mu) ** 2, axis=0, keepdims=True)
    t = (t - mu) * lax.rsqrt(var + _EPS) * g[...] + bb[...]
    t = jnp.maximum(t, 0.0)
    hh = jnp.maximum(
        jnp.dot(t, v1[...], preferred_element_type=jnp.float32) + c1[...], 0.0)
    out[...] = (jnp.dot(hh, v2[...], preferred_element_type=jnp.float32)
                + c2[...])


_tc_head = pl.pallas_call(
    _tc_head_body,
    out_shape=jax.ShapeDtypeStruct((_B, _OUT), jnp.float32),
)


def kernel(x_ori, gamora0, gamora1, gamora2, edge_index, Wl, bl, Wr, bn_g,
           bn_b, m1_W1, m1_b1, m1_W2, m1_b2, bn2_g, bn2_b, m2_W1, m2_b1,
           m2_W2, m2_b2):
    x = _tc_concat4(x_ori, gamora0, gamora1, gamora2)

    packed = jnp.concatenate([_tc_pack(edge_index),
                              jnp.asarray(_PAD_PACKED)])
    zerosd = jnp.zeros((_RPT, _DW), jnp.float32)
    ones = jnp.ones((_C, _DW), jnp.float32)

    degw = _sc_deg(packed, zerosd, ones)
    degn = degw[:, :_N, 0:1]
    for i in range(_L):
        aggp = _sc_agg(x, packed)
        yr = _tc_pre(x, Wr[i], bl[i].reshape(1, _H))
        x = _tc_post(aggp, yr, degn, Wl[i], bn_g[i].reshape(1, _H),
                     bn_b[i].reshape(1, _H))

    x2d = x.reshape(_N * _H // _MNN, _MNN)
    return _tc_head(x2d, m1_W1, m1_b1.reshape(1, _H), m1_W2.reshape(1, _H),
                    m1_b2.reshape(1, 1), bn2_g.reshape(1, _H),
                    bn2_b.reshape(1, _H), m2_W1, m2_b1.reshape(1, _H),
                    m2_W2, m2_b2.reshape(1, _OUT))
